# Initial kernel scaffold; baseline (speedup 1.0000x reference)
import jax
import jax.numpy as jnp
from jax.experimental import pallas as pl


def kernel(context_win, memory_win, embeddings_table):
    raise NotImplementedError("write your pallas kernel here")


# speedup vs baseline: 42.0139x; 42.0139x over previous
import functools

import jax
import jax.numpy as jnp
from jax import lax
from jax.experimental import pallas as pl
from jax.experimental.pallas import tpu as pltpu
from jax.experimental.pallas import tpu_sc as plsc

N_HEADS = 16
MAX_WINDOW = 2048
ROW_PAD = 4112
SHIFT_LEN = 4096
T_PER_CORE = MAX_WINDOW // 2
GROUP = 16


def _sc_body(table_hbm, out_hbm, row_v, *shifts_and_sem):
    shifts_v = shifts_and_sem[:8]
    sem = shifts_and_sem[8]
    c = lax.axis_index("c")
    s = lax.axis_index("s")
    h = s
    t0 = c * T_PER_CORE

    pltpu.sync_copy(table_hbm.at[h], row_v)

    def silu_step(i, carry):
        x = row_v[pl.ds(i * 16, 16)]
        row_v[pl.ds(i * 16, 16)] = x / (1.0 + jnp.exp(-x))
        return carry

    lax.fori_loop(0, ROW_PAD // 16, silu_step, 0)

    for r in range(8):
        def shift_step(i, carry, r=r):
            shifts_v[r][pl.ds(i * 16, 16)] = row_v[pl.ds(r + i * 16, 16)]
            return carry

        lax.fori_loop(0, SHIFT_LEN // 16, shift_step, 0)

    for r in range(8):
        roff = (8 - r) % 8
        iters = T_PER_CORE // 8

        def group_step(g, carry, r=r, roff=roff):
            copies = []
            for j in range(GROUP):
                t = t0 + roff + 8 * (g * GROUP + j)
                q8 = pl.multiple_of(MAX_WINDOW - t - r, 8)
                dst = pl.multiple_of((h * MAX_WINDOW + t) * MAX_WINDOW, 8)
                copies.append(
                    pltpu.async_copy(
                        shifts_v[r].at[pl.ds(q8, MAX_WINDOW)],
                        out_hbm.at[pl.ds(dst, MAX_WINDOW)],
                        sem,
                    )
                )
            for cp in copies:
                cp.wait()
            return carry

        lax.fori_loop(0, iters // GROUP, group_step, 0)


def kernel(context_win, memory_win, embeddings_table):
    del context_win, memory_win
    table_t = jnp.transpose(embeddings_table)
    table_t = jnp.pad(table_t, ((0, 0), (0, ROW_PAD - table_t.shape[1])))

    mesh = plsc.VectorSubcoreMesh(core_axis_name="c", subcore_axis_name="s")
    run = functools.partial(
        pl.kernel,
        mesh=mesh,
        out_type=jax.ShapeDtypeStruct(
            (N_HEADS * MAX_WINDOW * MAX_WINDOW,), jnp.float32
        ),
        scratch_types=[
            pltpu.VMEM((ROW_PAD,), jnp.float32),
            *[pltpu.VMEM((SHIFT_LEN,), jnp.float32) for _ in range(8)],
            pltpu.SemaphoreType.DMA,
        ],
    )(_sc_body)
    return jnp.reshape(run(table_t), (N_HEADS, MAX_WINDOW, MAX_WINDOW))
